# baseline (speedup 1.0000x reference)
import jax
import jax.numpy as jnp
from jax.experimental import pallas as pl

N = 8
K = 1024
K1 = K + 1
DIM = 1024
E_DIM = DIM // N
BETA = 0.25
B = 8192
BLK = 256


def _body(z_ref, ef_ref, et_ref, loss_ref, zq_ref, oh_ref, idx_ref):
    i = pl.program_id(0)

    part = 0.0
    for n in range(N):
        zr = z_ref[:, n * E_DIM:(n + 1) * E_DIM]
        et = et_ref[n]

        z_sq = jnp.sum(zr * zr, axis=1, keepdims=True)
        e_sq = jnp.sum(et * et, axis=0, keepdims=True)
        cross = jnp.dot(zr, et, preferred_element_type=jnp.float32)
        d = jnp.sqrt(jnp.maximum((z_sq + e_sq) - 2.0 * cross, 0.0))

        m = jnp.min(d, axis=1, keepdims=True)
        ii = jax.lax.broadcasted_iota(jnp.int32, d.shape, 1)
        amin = jnp.min(jnp.where(d == m, ii, K1), axis=1, keepdims=True)
        idx_ref[:, n:n + 1] = amin

        oh = (ii == amin).astype(jnp.float32)
        oh_ref[:, n, :] = oh
        zq = jnp.dot(oh, ef_ref[n], preferred_element_type=jnp.float32)
        zq_ref[:, n * E_DIM:(n + 1) * E_DIM] = zq

        diff = zq - zr
        part += jnp.sum(diff * diff)

    @pl.when(i == 0)
    def _init():
        loss_ref[...] = jnp.zeros((1, 1), jnp.float32)

    loss_ref[...] += jnp.reshape(part, (1, 1))


def kernel(z, embedding):
    pad = jnp.zeros((N, 1, E_DIM), dtype=embedding.dtype)
    emb_full = jnp.concatenate([pad, embedding], axis=1)
    emb_full_t = emb_full.transpose(0, 2, 1)

    grid = (B // BLK,)
    loss2d, zq, oh, idx = pl.pallas_call(
        _body,
        grid=grid,
        in_specs=[
            pl.BlockSpec((BLK, DIM), lambda i: (i, 0)),
            pl.BlockSpec((N, K1, E_DIM), lambda i: (0, 0, 0)),
            pl.BlockSpec((N, E_DIM, K1), lambda i: (0, 0, 0)),
        ],
        out_specs=[
            pl.BlockSpec((1, 1), lambda i: (0, 0)),
            pl.BlockSpec((BLK, DIM), lambda i: (i, 0)),
            pl.BlockSpec((BLK, N, K1), lambda i: (i, 0, 0)),
            pl.BlockSpec((BLK, N), lambda i: (i, 0)),
        ],
        out_shape=[
            jax.ShapeDtypeStruct((1, 1), jnp.float32),
            jax.ShapeDtypeStruct((B, DIM), jnp.float32),
            jax.ShapeDtypeStruct((B, N, K1), jnp.float32),
            jax.ShapeDtypeStruct((B, N), jnp.int32),
        ],
    )(z, emb_full, emb_full_t)

    mean_sq = loss2d[0, 0] / (B * DIM)
    loss = mean_sq + BETA * mean_sq
    min_encodings = oh
    min_encoding_indices = idx.reshape(B, N, 1)
    return (loss, zq, 0, min_encodings, min_encoding_indices)


# speedup vs baseline: 2.0664x; 2.0664x over previous
import jax
import jax.numpy as jnp
from jax.experimental import pallas as pl

N = 8
K = 1024
K1 = K + 1
DIM = 1024
E_DIM = DIM // N
BETA = 0.25
B = 8192
BLK = 256


def _body(z_ref, ef_ref, et_ref, loss_ref, zq_ref, oh_ref, idx_ref):
    i = pl.program_id(0)

    ii = jax.lax.broadcasted_iota(jnp.int32, (BLK, K1), 1)
    part = 0.0
    amins = []
    for n in range(N):
        zr = z_ref[:, n * E_DIM:(n + 1) * E_DIM]
        et = et_ref[n]

        z_sq = jnp.sum(zr * zr, axis=1, keepdims=True)
        e_sq = jnp.sum(et * et, axis=0, keepdims=True)
        cross = jnp.dot(zr, et, preferred_element_type=jnp.float32)
        d = jnp.sqrt(jnp.maximum((z_sq + e_sq) - 2.0 * cross, 0.0))

        m = jnp.min(d, axis=1, keepdims=True)
        amin = jnp.min(jnp.where(d == m, ii, K1), axis=1, keepdims=True)
        amins.append(amin)
        idx_ref[:, n:n + 1] = amin

        oh = (ii == amin).astype(jnp.float32)
        zq = jnp.dot(oh, ef_ref[n], preferred_element_type=jnp.float32)
        zq_ref[:, n * E_DIM:(n + 1) * E_DIM] = zq

        diff = zq - zr
        part += jnp.sum(diff * diff)

    idx_cols = jnp.concatenate(amins, axis=1)
    idx_rows = jnp.transpose(idx_cols)
    idx_rows3 = idx_rows[None]
    kk = jax.lax.broadcasted_iota(jnp.int32, (K1, N, BLK), 0)
    oh_ref[...] = (kk == idx_rows3).astype(jnp.float32)

    @pl.when(i == 0)
    def _init():
        loss_ref[...] = jnp.zeros((1, 1), jnp.float32)

    loss_ref[...] += jnp.reshape(part, (1, 1))


def kernel(z, embedding):
    pad = jnp.zeros((N, 1, E_DIM), dtype=embedding.dtype)
    emb_full = jnp.concatenate([pad, embedding], axis=1)
    emb_full_t = emb_full.transpose(0, 2, 1)

    grid = (B // BLK,)
    loss2d, zq, oh_t, idx = pl.pallas_call(
        _body,
        grid=grid,
        in_specs=[
            pl.BlockSpec((BLK, DIM), lambda i: (i, 0)),
            pl.BlockSpec((N, K1, E_DIM), lambda i: (0, 0, 0)),
            pl.BlockSpec((N, E_DIM, K1), lambda i: (0, 0, 0)),
        ],
        out_specs=[
            pl.BlockSpec((1, 1), lambda i: (0, 0)),
            pl.BlockSpec((BLK, DIM), lambda i: (i, 0)),
            pl.BlockSpec((K1, N, BLK), lambda i: (0, 0, i)),
            pl.BlockSpec((BLK, N), lambda i: (i, 0)),
        ],
        out_shape=[
            jax.ShapeDtypeStruct((1, 1), jnp.float32),
            jax.ShapeDtypeStruct((B, DIM), jnp.float32),
            jax.ShapeDtypeStruct((K1, N, B), jnp.float32),
            jax.ShapeDtypeStruct((B, N), jnp.int32),
        ],
    )(z, emb_full, emb_full_t)

    mean_sq = loss2d[0, 0] / (B * DIM)
    loss = mean_sq + BETA * mean_sq
    min_encodings = jnp.transpose(oh_t, (2, 1, 0))
    min_encoding_indices = idx.reshape(B, N, 1)
    return (loss, zq, 0, min_encodings, min_encoding_indices)


# speedup vs baseline: 2.8268x; 1.3680x over previous
import jax
import jax.numpy as jnp
from jax.experimental import pallas as pl
from jax.experimental.pallas import tpu as pltpu

N = 8
K = 1024
K1 = K + 1
DIM = 1024
E_DIM = DIM // N
BETA = 0.25
B = 8192
BLK = 256


def _body(z_ref, ef_ref, et_ref, loss_ref, zq_ref, oh_ref, idx_ref, t_ref):
    i = pl.program_id(0)

    ii = jax.lax.broadcasted_iota(jnp.int32, (BLK, K1), 1)

    m2s = []
    for n in range(N):
        zr = z_ref[:, n * E_DIM:(n + 1) * E_DIM]
        et = et_ref[n]
        z_sq = jnp.sum(zr * zr, axis=1, keepdims=True)
        e_sq = jnp.sum(et * et, axis=0, keepdims=True)
        cross = jnp.dot(zr, et, preferred_element_type=jnp.float32)
        t = (z_sq + e_sq) - 2.0 * cross
        t_ref[n] = t
        m2s.append(jnp.min(t, axis=1, keepdims=True))

    m2 = jnp.concatenate(m2s, axis=1)
    sm = jnp.sqrt(jnp.maximum(m2, 0.0))
    smb = jax.lax.bitcast_convert_type(sm, jnp.int32)
    nxt = jax.lax.bitcast_convert_type(smb + 1, jnp.float32)
    mid = sm + 0.5 * (nxt - sm)
    A = mid * mid
    for _ in range(3):
        Ab = jax.lax.bitcast_convert_type(A, jnp.int32)
        Ap = jax.lax.bitcast_convert_type(Ab - 1, jnp.float32)
        A = jnp.where(jnp.sqrt(A) > sm, Ap, A)
    for _ in range(3):
        Ab = jax.lax.bitcast_convert_type(A, jnp.int32)
        An = jax.lax.bitcast_convert_type(Ab + 1, jnp.float32)
        A = jnp.where(jnp.sqrt(An) <= sm, An, A)
    T = jnp.where(m2 > 0.0, A, 0.0)

    part = 0.0
    amins = []
    for n in range(N):
        t = t_ref[n]
        amin = jnp.min(jnp.where(t <= T[:, n:n + 1], ii, K1),
                       axis=1, keepdims=True)
        amins.append(amin)
        idx_ref[:, n:n + 1] = amin

        oh = (ii == amin).astype(jnp.float32)
        zq = jnp.dot(oh, ef_ref[n], preferred_element_type=jnp.float32)
        zq_ref[:, n * E_DIM:(n + 1) * E_DIM] = zq

        zr = z_ref[:, n * E_DIM:(n + 1) * E_DIM]
        diff = zq - zr
        part += jnp.sum(diff * diff)

    idx_cols = jnp.concatenate(amins, axis=1)
    idx_rows = jnp.transpose(idx_cols)
    idx_rows3 = idx_rows[None]
    kk = jax.lax.broadcasted_iota(jnp.int32, (K1, N, BLK), 0)
    oh_ref[...] = (kk == idx_rows3).astype(jnp.float32)

    @pl.when(i == 0)
    def _init():
        loss_ref[...] = jnp.zeros((1, 1), jnp.float32)

    loss_ref[...] += jnp.reshape(part, (1, 1))


def kernel(z, embedding):
    pad = jnp.zeros((N, 1, E_DIM), dtype=embedding.dtype)
    emb_full = jnp.concatenate([pad, embedding], axis=1)
    emb_full_t = emb_full.transpose(0, 2, 1)

    grid = (B // BLK,)
    loss2d, zq, oh_t, idx = pl.pallas_call(
        _body,
        grid=grid,
        in_specs=[
            pl.BlockSpec((BLK, DIM), lambda i: (i, 0)),
            pl.BlockSpec((N, K1, E_DIM), lambda i: (0, 0, 0)),
            pl.BlockSpec((N, E_DIM, K1), lambda i: (0, 0, 0)),
        ],
        out_specs=[
            pl.BlockSpec((1, 1), lambda i: (0, 0)),
            pl.BlockSpec((BLK, DIM), lambda i: (i, 0)),
            pl.BlockSpec((K1, N, BLK), lambda i: (0, 0, i)),
            pl.BlockSpec((BLK, N), lambda i: (i, 0)),
        ],
        out_shape=[
            jax.ShapeDtypeStruct((1, 1), jnp.float32),
            jax.ShapeDtypeStruct((B, DIM), jnp.float32),
            jax.ShapeDtypeStruct((K1, N, B), jnp.float32),
            jax.ShapeDtypeStruct((B, N), jnp.int32),
        ],
        scratch_shapes=[pltpu.VMEM((N, BLK, K1), jnp.float32)],
    )(z, emb_full, emb_full_t)

    mean_sq = loss2d[0, 0] / (B * DIM)
    loss = mean_sq + BETA * mean_sq
    min_encodings = jnp.transpose(oh_t, (2, 1, 0))
    min_encoding_indices = idx.reshape(B, N, 1)
    return (loss, zq, 0, min_encodings, min_encoding_indices)
